# baseline (device time: 2211076 ns/iter reference)
import jax
import jax.numpy as jnp
from jax import lax
from jax.experimental import pallas as pl
from jax.experimental.pallas import tpu as pltpu

N_DEV = 32
B, Sq, Skv = 2, 512, 512
H_PER, Dh = 8, 64
D_MODEL = 768
ROWS = B * Sq
CHUNK = ROWS // N_DEV
CHUNKS_PER_B = Sq // CHUNK


def _body(x_ref, wq_ref, wo_ref, k_hbm, v_hbm, out_ref,
          out16, k_scr, v_scr, rs_recv,
          kv_sems, rs_send_sems, rs_recv_sems, ag_send_sems, ag_recv_sems):
    my = lax.axis_index("i")
    bf16 = jnp.bfloat16

    barrier = pltpu.get_barrier_semaphore()
    for j in range(1, N_DEV):
        pl.semaphore_signal(barrier, inc=1,
                            device_id=(lax.rem(my + j, N_DEV),),
                            device_id_type=pl.DeviceIdType.MESH)

    kv_copies = []
    for t, (hbm, scr) in enumerate(((k_hbm, k_scr), (v_hbm, v_scr))):
        c = pltpu.make_async_copy(
            hbm.at[:, pl.ds(my * H_PER * Dh, H_PER * Dh)],
            scr,
            kv_sems.at[t],
        )
        c.start()
        kv_copies.append(c)

    x16 = x_ref[...].astype(bf16)
    wq16 = (wq_ref[...] * 0.125).astype(bf16)
    wo16 = wo_ref[...].astype(bf16)

    qi = lax.broadcasted_iota(jnp.int32, (Sq, Skv), 0)
    ki = lax.broadcasted_iota(jnp.int32, (Sq, Skv), 1)
    mask = (jnp.abs(qi - ki) <= 128) | (ki < 32) | (qi < 32)
    neg = jnp.float32(-1e9)

    for c in kv_copies:
        c.wait()
    pl.semaphore_wait(barrier, N_DEV - 1)

    mine = pl.ds(my * CHUNK, CHUNK)

    q16 = jnp.dot(x16, wq16,
                  preferred_element_type=jnp.float32).astype(bf16)
    k16 = k_scr[...].astype(bf16)
    v16 = v_scr[...].astype(bf16)
    for b in range(B):
        rows = slice(b * Sq, (b + 1) * Sq)
        ctx_parts = []
        for h in range(H_PER):
            cols = slice(h * Dh, (h + 1) * Dh)
            s = lax.dot_general(q16[rows, cols], k16[rows, cols],
                                (((1,), (1,)), ((), ())),
                                preferred_element_type=jnp.float32)
            s = jnp.where(mask, s, neg)
            s = s - jnp.max(s, axis=1, keepdims=True)
            w = jnp.exp(s)
            inv = 1.0 / jnp.sum(w, axis=1, keepdims=True)
            ctx = jnp.dot(w.astype(bf16), v16[rows, cols],
                          preferred_element_type=jnp.float32)
            ctx_parts.append((ctx * inv).astype(bf16))
        ctx_b = jnp.concatenate(ctx_parts, axis=1)
        out16[rows, :] = jnp.dot(ctx_b, wo16,
                                 preferred_element_type=jnp.float32).astype(bf16)

        for d in range(b * CHUNKS_PER_B, (b + 1) * CHUNKS_PER_B):
            @pl.when(d != my)
            def _(d=d):
                off = lax.rem(my - d + N_DEV, N_DEV)
                pltpu.make_async_remote_copy(
                    src_ref=out16.at[pl.ds(d * CHUNK, CHUNK), :],
                    dst_ref=rs_recv.at[off],
                    send_sem=rs_send_sems.at[d],
                    recv_sem=rs_recv_sems.at[off],
                    device_id=(d,),
                    device_id_type=pl.DeviceIdType.MESH,
                ).start()

    for j in range(1, N_DEV):
        pltpu.make_async_remote_copy(
            src_ref=rs_recv.at[j], dst_ref=rs_recv.at[j],
            send_sem=rs_send_sems.at[0], recv_sem=rs_recv_sems.at[j],
            device_id=(my,), device_id_type=pl.DeviceIdType.MESH,
        ).wait_recv()
    total = (out16[mine, :].astype(jnp.float32)
             + jnp.sum(rs_recv[1:].astype(jnp.float32), axis=0))
    out16[mine, :] = total.astype(bf16)
    for d in range(N_DEV):
        @pl.when(d != my)
        def _(d=d):
            pltpu.make_async_remote_copy(
                src_ref=out16.at[mine, :],
                dst_ref=out16.at[mine, :],
                send_sem=ag_send_sems.at[d],
                recv_sem=ag_recv_sems.at[my],
                device_id=(d,),
                device_id_type=pl.DeviceIdType.MESH,
            ).start()

    for j in range(N_DEV):
        @pl.when(j != my)
        def _(j=j):
            pltpu.make_async_remote_copy(
                src_ref=out16.at[pl.ds(j * CHUNK, CHUNK), :],
                dst_ref=out16.at[pl.ds(j * CHUNK, CHUNK), :],
                send_sem=ag_send_sems.at[j],
                recv_sem=ag_recv_sems.at[j],
                device_id=(my,), device_id_type=pl.DeviceIdType.MESH,
            ).wait_recv()

    out_ref[...] = out16[...].astype(jnp.float32)

    for d in range(N_DEV):
        @pl.when(d != my)
        def _(d=d):
            pltpu.make_async_remote_copy(
                src_ref=out16.at[pl.ds(d * CHUNK, CHUNK), :],
                dst_ref=rs_recv.at[0],
                send_sem=rs_send_sems.at[d],
                recv_sem=rs_recv_sems.at[0],
                device_id=(d,), device_id_type=pl.DeviceIdType.MESH,
            ).wait_send()
            pltpu.make_async_remote_copy(
                src_ref=out16.at[mine, :],
                dst_ref=rs_recv.at[0],
                send_sem=ag_send_sems.at[d],
                recv_sem=rs_recv_sems.at[0],
                device_id=(d,), device_id_type=pl.DeviceIdType.MESH,
            ).wait_send()


def kernel(x, Wq, K_ext, V_ext, Wo):
    x2 = x.reshape(ROWS, D_MODEL)
    k2 = K_ext.reshape(B * Skv, 256 * Dh)
    v2 = V_ext.reshape(B * Skv, 256 * Dh)
    out2 = pl.pallas_call(
        _body,
        out_shape=jax.ShapeDtypeStruct((ROWS, D_MODEL), jnp.float32),
        in_specs=[
            pl.BlockSpec(memory_space=pltpu.VMEM),
            pl.BlockSpec(memory_space=pltpu.VMEM),
            pl.BlockSpec(memory_space=pltpu.VMEM),
            pl.BlockSpec(memory_space=pltpu.MemorySpace.HBM),
            pl.BlockSpec(memory_space=pltpu.MemorySpace.HBM),
        ],
        out_specs=pl.BlockSpec(memory_space=pltpu.VMEM),
        scratch_shapes=[
            pltpu.VMEM((ROWS, D_MODEL), jnp.bfloat16),
            pltpu.VMEM((B * Skv, H_PER * Dh), jnp.float32),
            pltpu.VMEM((B * Skv, H_PER * Dh), jnp.float32),
            pltpu.VMEM((N_DEV, CHUNK, D_MODEL), jnp.bfloat16),
            pltpu.SemaphoreType.DMA((2,)),
            pltpu.SemaphoreType.DMA((N_DEV,)),
            pltpu.SemaphoreType.DMA((N_DEV,)),
            pltpu.SemaphoreType.DMA((N_DEV,)),
            pltpu.SemaphoreType.DMA((N_DEV,)),
        ],
        compiler_params=pltpu.CompilerParams(collective_id=0),
    )(x2, Wq, Wo, k2, v2)
    return out2.reshape(B, Sq, D_MODEL)


# device time: 256605 ns/iter; 8.6167x vs baseline; 8.6167x over previous
import jax
import jax.numpy as jnp
from jax import lax
from jax.experimental import pallas as pl
from jax.experimental.pallas import tpu as pltpu

N_DEV = 32
B, Sq, Skv = 2, 512, 512
H_PER, Dh = 8, 64
D_MODEL = 768
ROWS = B * Sq
CHUNK = ROWS // N_DEV
CHUNKS_PER_B = Sq // CHUNK


def _body(x_ref, wq_ref, wo_ref, k_hbm, v_hbm, out_ref,
          out16, k_scr, v_scr, rs_recv,
          kv_sems, rs_send_sems, rs_recv_sems, ag_send_sems, ag_recv_sems):
    my = lax.axis_index("i")
    bf16 = jnp.bfloat16

    barrier = pltpu.get_barrier_semaphore()
    for j in range(1, N_DEV):
        pl.semaphore_signal(barrier, inc=1,
                            device_id=(lax.rem(my + j, N_DEV),),
                            device_id_type=pl.DeviceIdType.MESH)

    kv_copies = []
    for t, (hbm, scr) in enumerate(((k_hbm, k_scr), (v_hbm, v_scr))):
        for b in range(B):
            for h in range(H_PER):
                c = pltpu.make_async_copy(
                    hbm.at[b, :, my * H_PER + h, :],
                    scr.at[b, h],
                    kv_sems.at[t, b, h],
                )
                c.start()
                kv_copies.append(c)

    x16 = x_ref[...].astype(bf16)
    wq16 = (wq_ref[...] * 0.125).astype(bf16)
    wo16 = wo_ref[...].astype(bf16)

    qi = lax.broadcasted_iota(jnp.int32, (Sq, Skv), 0)
    ki = lax.broadcasted_iota(jnp.int32, (Sq, Skv), 1)
    mask = (jnp.abs(qi - ki) <= 128) | (ki < 32) | (qi < 32)
    neg = jnp.float32(-1e9)

    for c in kv_copies:
        c.wait()
    pl.semaphore_wait(barrier, N_DEV - 1)

    mine = pl.ds(my * CHUNK, CHUNK)

    for b in range(B):
        xb = x16[b * Sq:(b + 1) * Sq, :]
        acc = jnp.zeros((Sq, D_MODEL), jnp.float32)
        for h in range(H_PER):
            qbh = jnp.dot(xb, wq16[:, h * Dh:(h + 1) * Dh],
                          preferred_element_type=jnp.float32).astype(bf16)
            s = lax.dot_general(qbh, k_scr[b, h].astype(bf16),
                                (((1,), (1,)), ((), ())),
                                preferred_element_type=jnp.float32)
            s = jnp.where(mask, s, neg)
            s = s - jnp.max(s, axis=1, keepdims=True)
            w = jnp.exp(s)
            inv = 1.0 / jnp.sum(w, axis=1, keepdims=True)
            ctx = jnp.dot(w.astype(bf16), v_scr[b, h].astype(bf16),
                          preferred_element_type=jnp.float32)
            ctx = (ctx * inv).astype(bf16)
            acc = acc + jnp.dot(ctx, wo16[h * Dh:(h + 1) * Dh, :],
                                preferred_element_type=jnp.float32)
        out16[b * Sq:(b + 1) * Sq, :] = acc.astype(bf16)

        for d in range(b * CHUNKS_PER_B, (b + 1) * CHUNKS_PER_B):
            @pl.when(d != my)
            def _(d=d):
                off = lax.rem(my - d + N_DEV, N_DEV)
                pltpu.make_async_remote_copy(
                    src_ref=out16.at[pl.ds(d * CHUNK, CHUNK), :],
                    dst_ref=rs_recv.at[off],
                    send_sem=rs_send_sems.at[d],
                    recv_sem=rs_recv_sems.at[off],
                    device_id=(d,),
                    device_id_type=pl.DeviceIdType.MESH,
                ).start()

    for j in range(1, N_DEV):
        pltpu.make_async_remote_copy(
            src_ref=rs_recv.at[j], dst_ref=rs_recv.at[j],
            send_sem=rs_send_sems.at[0], recv_sem=rs_recv_sems.at[j],
            device_id=(my,), device_id_type=pl.DeviceIdType.MESH,
        ).wait_recv()
    total = (out16[mine, :].astype(jnp.float32)
             + jnp.sum(rs_recv[1:].astype(jnp.float32), axis=0))
    out16[mine, :] = total.astype(bf16)
    for d in range(N_DEV):
        @pl.when(d != my)
        def _(d=d):
            pltpu.make_async_remote_copy(
                src_ref=out16.at[mine, :],
                dst_ref=out16.at[mine, :],
                send_sem=ag_send_sems.at[d],
                recv_sem=ag_recv_sems.at[my],
                device_id=(d,),
                device_id_type=pl.DeviceIdType.MESH,
            ).start()

    for j in range(N_DEV):
        @pl.when(j != my)
        def _(j=j):
            pltpu.make_async_remote_copy(
                src_ref=out16.at[pl.ds(j * CHUNK, CHUNK), :],
                dst_ref=out16.at[pl.ds(j * CHUNK, CHUNK), :],
                send_sem=ag_send_sems.at[j],
                recv_sem=ag_recv_sems.at[j],
                device_id=(my,), device_id_type=pl.DeviceIdType.MESH,
            ).wait_recv()

    out_ref[...] = out16[...].astype(jnp.float32)

    for d in range(N_DEV):
        @pl.when(d != my)
        def _(d=d):
            pltpu.make_async_remote_copy(
                src_ref=out16.at[pl.ds(d * CHUNK, CHUNK), :],
                dst_ref=rs_recv.at[0],
                send_sem=rs_send_sems.at[d],
                recv_sem=rs_recv_sems.at[0],
                device_id=(d,), device_id_type=pl.DeviceIdType.MESH,
            ).wait_send()
            pltpu.make_async_remote_copy(
                src_ref=out16.at[mine, :],
                dst_ref=rs_recv.at[0],
                send_sem=ag_send_sems.at[d],
                recv_sem=rs_recv_sems.at[0],
                device_id=(d,), device_id_type=pl.DeviceIdType.MESH,
            ).wait_send()


def kernel(x, Wq, K_ext, V_ext, Wo):
    x2 = x.reshape(ROWS, D_MODEL)
    out2 = pl.pallas_call(
        _body,
        out_shape=jax.ShapeDtypeStruct((ROWS, D_MODEL), jnp.float32),
        in_specs=[
            pl.BlockSpec(memory_space=pltpu.VMEM),
            pl.BlockSpec(memory_space=pltpu.VMEM),
            pl.BlockSpec(memory_space=pltpu.VMEM),
            pl.BlockSpec(memory_space=pltpu.MemorySpace.HBM),
            pl.BlockSpec(memory_space=pltpu.MemorySpace.HBM),
        ],
        out_specs=pl.BlockSpec(memory_space=pltpu.VMEM),
        scratch_shapes=[
            pltpu.VMEM((ROWS, D_MODEL), jnp.bfloat16),
            pltpu.VMEM((B, H_PER, Skv, Dh), jnp.float32),
            pltpu.VMEM((B, H_PER, Skv, Dh), jnp.float32),
            pltpu.VMEM((N_DEV, CHUNK, D_MODEL), jnp.bfloat16),
            pltpu.SemaphoreType.DMA((2, B, H_PER)),
            pltpu.SemaphoreType.DMA((N_DEV,)),
            pltpu.SemaphoreType.DMA((N_DEV,)),
            pltpu.SemaphoreType.DMA((N_DEV,)),
            pltpu.SemaphoreType.DMA((N_DEV,)),
        ],
        compiler_params=pltpu.CompilerParams(collective_id=0),
    )(x2, Wq, Wo, K_ext, V_ext)
    return out2.reshape(B, Sq, D_MODEL)
